# Initial kernel scaffold; baseline (speedup 1.0000x reference)
import jax
import jax.numpy as jnp
from jax.experimental import pallas as pl


def kernel(embed, src, rel, dst, labels, w_relation):
    raise NotImplementedError("write your pallas kernel here")


# speedup vs baseline: 1.7819x; 1.7819x over previous
import functools

import jax
import jax.numpy as jnp
from jax import lax
from jax.experimental import pallas as pl
from jax.experimental.pallas import tpu as pltpu
from jax.experimental.pallas import tpu_sc as plsc

H = 128
LANES = 16
CHUNK = 128
NUM_R = 200
REG = 0.01


def _sc_parts_body(nc, ns, chunks_per_w,
                   embed_hbm, src_hbm, dst_hbm, rel_hbm, w_hbm,
                   out_hbm,
                   sidx, didx, ridx, srows, orows, wrows, parts):
    wid = lax.axis_index("s") * nc + lax.axis_index("c")
    edges_per_w = chunks_per_w * CHUNK
    base_w = wid * edges_per_w

    def chunk_body(c, _):
        base = base_w + c * CHUNK
        pltpu.sync_copy(src_hbm.at[pl.ds(base, CHUNK)], sidx)
        pltpu.sync_copy(dst_hbm.at[pl.ds(base, CHUNK)], didx)
        pltpu.sync_copy(rel_hbm.at[pl.ds(base, CHUNK)], ridx)
        pltpu.sync_copy(embed_hbm.at[sidx], srows)
        pltpu.sync_copy(embed_hbm.at[didx], orows)
        pltpu.sync_copy(w_hbm.at[ridx], wrows)

        def edge_body(e, _):
            acc = (srows[e, pl.ds(0, LANES)] * orows[e, pl.ds(0, LANES)]
                   * wrows[e, pl.ds(0, LANES)])
            for j in range(1, H // LANES):
                acc = acc + (srows[e, pl.ds(j * LANES, LANES)]
                             * orows[e, pl.ds(j * LANES, LANES)]
                             * wrows[e, pl.ds(j * LANES, LANES)])
            parts[e, pl.ds(0, LANES)] = acc
            return 0

        lax.fori_loop(0, CHUNK, edge_body, 0)
        pltpu.sync_copy(parts, out_hbm.at[pl.ds(base, CHUNK)])
        return 0

    lax.fori_loop(0, chunks_per_w, chunk_body, 0)


def _sc_parts(embed, src, dst, rel, w_relation):
    e_pad = src.shape[0]
    info = plsc.get_sparse_core_info()
    nc, ns = info.num_cores, info.num_subcores
    nw = nc * ns
    chunks_per_w = e_pad // (nw * CHUNK)
    mesh = plsc.VectorSubcoreMesh(core_axis_name="c", subcore_axis_name="s")
    body = functools.partial(_sc_parts_body, nc, ns, chunks_per_w)
    return pl.kernel(
        body,
        out_type=jax.ShapeDtypeStruct((e_pad, LANES), jnp.float32),
        mesh=mesh,
        scratch_types=[
            pltpu.VMEM((CHUNK,), jnp.int32),
            pltpu.VMEM((CHUNK,), jnp.int32),
            pltpu.VMEM((CHUNK,), jnp.int32),
            pltpu.VMEM((CHUNK, H), jnp.float32),
            pltpu.VMEM((CHUNK, H), jnp.float32),
            pltpu.VMEM((CHUNK, H), jnp.float32),
            pltpu.VMEM((CHUNK, LANES), jnp.float32),
        ],
    )(embed, src, dst, rel, w_relation)


def _tc_loss_kernel(n_valid, br, parts_ref, labels_ref, embed_ref, w_ref,
                    out_ref):
    i = pl.program_id(0)

    @pl.when(i == 0)
    def _init():
        n_emb = embed_ref.shape[0] * embed_ref.shape[1]
        n_w = w_ref.shape[0] * w_ref.shape[1]
        reg = (jnp.sum(embed_ref[...] ** 2) / n_emb
               + jnp.sum(w_ref[...] ** 2) / n_w)
        out_ref[...] = jnp.reshape(REG * reg, (1, 1))

    p = parts_ref[...]
    d_iota = lax.broadcasted_iota(jnp.int32, (H, 8), 0)
    k_iota = lax.broadcasted_iota(jnp.int32, (H, 8), 1)
    fold = (d_iota // LANES == k_iota).astype(jnp.float32)
    s = jnp.dot(p, fold, preferred_element_type=jnp.float32)
    l = labels_ref[...]
    bce = jnp.maximum(s, 0.0) - s * l + jnp.log1p(jnp.exp(-jnp.abs(s)))
    row_i = lax.broadcasted_iota(jnp.int32, (br, 8), 0)
    k_i = lax.broadcasted_iota(jnp.int32, (br, 8), 1)
    edge_i = (i * br + row_i) * 8 + k_i
    bce = jnp.where(edge_i < n_valid, bce, 0.0)
    out_ref[...] += jnp.reshape(jnp.sum(bce) / n_valid, (1, 1))


def _tc_loss(n_valid, parts2, labels8, embed, w_relation):
    rows = parts2.shape[0]
    nb = 16
    while rows % nb:
        nb //= 2
    br = rows // nb
    kfn = functools.partial(_tc_loss_kernel, n_valid, br)
    return pl.pallas_call(
        kfn,
        grid=(nb,),
        in_specs=[
            pl.BlockSpec((br, H), lambda i: (i, 0)),
            pl.BlockSpec((br, 8), lambda i: (i, 0)),
            pl.BlockSpec(embed.shape, lambda i: (0, 0)),
            pl.BlockSpec(w_relation.shape, lambda i: (0, 0)),
        ],
        out_specs=pl.BlockSpec((1, 1), lambda i: (0, 0)),
        out_shape=jax.ShapeDtypeStruct((1, 1), jnp.float32),
    )(parts2, labels8, embed, w_relation)


def kernel(embed, src, rel, dst, labels, w_relation):
    e = src.shape[0]
    src = src.astype(jnp.int32)
    dst = dst.astype(jnp.int32)
    rel = rel.astype(jnp.int32)
    granule = 32 * CHUNK
    e_pad = ((e + granule - 1) // granule) * granule
    pad = e_pad - e
    src_p = jnp.pad(src, (0, pad))
    dst_p = jnp.pad(dst, (0, pad))
    rel_p = jnp.pad(rel, (0, pad))
    parts = _sc_parts(embed, src_p, dst_p, rel_p, w_relation)
    parts2 = parts.reshape(-1, H)
    labels8 = jnp.pad(labels, (0, pad)).reshape(-1, 8)
    loss = _tc_loss(e, parts2, labels8, embed, w_relation)
    return loss.reshape(())


# speedup vs baseline: 1.8735x; 1.0514x over previous
import functools

import jax
import jax.numpy as jnp
from jax import lax
from jax.experimental import pallas as pl
from jax.experimental.pallas import tpu as pltpu
from jax.experimental.pallas import tpu_sc as plsc

H = 128
LANES = 16
CHUNK = 128
NUM_R = 200
REG = 0.01


def _sc_parts_body(nc, ns, chunks_per_w,
                   embed_hbm, src_hbm, dst_hbm, rel_hbm, w_hbm,
                   out_hbm,
                   sidx0, didx0, ridx0, srows0, orows0, wrows0,
                   sidx1, didx1, ridx1, srows1, orows1, wrows1,
                   parts, sem0, sem1):
    wid = lax.axis_index("s") * nc + lax.axis_index("c")
    edges_per_w = chunks_per_w * CHUNK
    base_w = wid * edges_per_w
    bufs = ((sidx0, didx0, ridx0, srows0, orows0, wrows0, sem0),
            (sidx1, didx1, ridx1, srows1, orows1, wrows1, sem1))

    def fire(c, b):
        sidx, didx, ridx, srows, orows, wrows, sem = bufs[b]
        base = base_w + c * CHUNK
        pltpu.sync_copy(src_hbm.at[pl.ds(base, CHUNK)], sidx)
        pltpu.sync_copy(dst_hbm.at[pl.ds(base, CHUNK)], didx)
        pltpu.sync_copy(rel_hbm.at[pl.ds(base, CHUNK)], ridx)
        pltpu.async_copy(embed_hbm.at[sidx], srows, sem)
        pltpu.async_copy(embed_hbm.at[didx], orows, sem)
        pltpu.async_copy(w_hbm.at[ridx], wrows, sem)

    def drain(b):
        sidx, didx, ridx, srows, orows, wrows, sem = bufs[b]
        pltpu.make_async_copy(embed_hbm.at[sidx], srows, sem).wait()
        pltpu.make_async_copy(embed_hbm.at[didx], orows, sem).wait()
        pltpu.make_async_copy(w_hbm.at[ridx], wrows, sem).wait()

    def compute(c, b):
        sidx, didx, ridx, srows, orows, wrows, sem = bufs[b]
        base = base_w + c * CHUNK

        def edge_body(e, _):
            acc = (srows[e, pl.ds(0, LANES)] * orows[e, pl.ds(0, LANES)]
                   * wrows[e, pl.ds(0, LANES)])
            for j in range(1, H // LANES):
                acc = acc + (srows[e, pl.ds(j * LANES, LANES)]
                             * orows[e, pl.ds(j * LANES, LANES)]
                             * wrows[e, pl.ds(j * LANES, LANES)])
            parts[e, pl.ds(0, LANES)] = acc
            return 0

        lax.fori_loop(0, CHUNK, edge_body, 0)
        pltpu.sync_copy(parts, out_hbm.at[pl.ds(base, CHUNK)])

    fire(0, 0)

    def pair_body(k, _):
        c = 2 * k
        fire(c + 1, 1)
        drain(0)
        compute(c, 0)

        @pl.when(c + 2 < chunks_per_w)
        def _():
            fire(c + 2, 0)

        drain(1)
        compute(c + 1, 1)
        return 0

    lax.fori_loop(0, chunks_per_w // 2, pair_body, 0)


def _sc_parts(embed, src, dst, rel, w_relation):
    e_pad = src.shape[0]
    info = plsc.get_sparse_core_info()
    nc, ns = info.num_cores, info.num_subcores
    nw = nc * ns
    chunks_per_w = e_pad // (nw * CHUNK)
    mesh = plsc.VectorSubcoreMesh(core_axis_name="c", subcore_axis_name="s")
    body = functools.partial(_sc_parts_body, nc, ns, chunks_per_w)
    buf_set = [
        pltpu.VMEM((CHUNK,), jnp.int32),
        pltpu.VMEM((CHUNK,), jnp.int32),
        pltpu.VMEM((CHUNK,), jnp.int32),
        pltpu.VMEM((CHUNK, H), jnp.float32),
        pltpu.VMEM((CHUNK, H), jnp.float32),
        pltpu.VMEM((CHUNK, H), jnp.float32),
    ]
    return pl.kernel(
        body,
        out_type=jax.ShapeDtypeStruct((e_pad, LANES), jnp.float32),
        mesh=mesh,
        scratch_types=buf_set + buf_set + [
            pltpu.VMEM((CHUNK, LANES), jnp.float32),
            pltpu.SemaphoreType.DMA,
            pltpu.SemaphoreType.DMA,
        ],
    )(embed, src, dst, rel, w_relation)


def _tc_loss_kernel(n_valid, br, parts_ref, labels_ref, embed_ref, w_ref,
                    out_ref):
    i = pl.program_id(0)

    @pl.when(i == 0)
    def _init():
        n_emb = embed_ref.shape[0] * embed_ref.shape[1]
        n_w = w_ref.shape[0] * w_ref.shape[1]
        reg = (jnp.sum(embed_ref[...] ** 2) / n_emb
               + jnp.sum(w_ref[...] ** 2) / n_w)
        out_ref[...] = jnp.reshape(REG * reg, (1, 1))

    p = parts_ref[...]
    d_iota = lax.broadcasted_iota(jnp.int32, (H, 8), 0)
    k_iota = lax.broadcasted_iota(jnp.int32, (H, 8), 1)
    fold = (d_iota // LANES == k_iota).astype(jnp.float32)
    s = jnp.dot(p, fold, preferred_element_type=jnp.float32)
    l = labels_ref[...]
    bce = jnp.maximum(s, 0.0) - s * l + jnp.log1p(jnp.exp(-jnp.abs(s)))
    row_i = lax.broadcasted_iota(jnp.int32, (br, 8), 0)
    k_i = lax.broadcasted_iota(jnp.int32, (br, 8), 1)
    edge_i = (i * br + row_i) * 8 + k_i
    bce = jnp.where(edge_i < n_valid, bce, 0.0)
    out_ref[...] += jnp.reshape(jnp.sum(bce) / n_valid, (1, 1))


def _tc_loss(n_valid, parts2, labels8, embed, w_relation):
    rows = parts2.shape[0]
    nb = 16
    while rows % nb:
        nb //= 2
    br = rows // nb
    kfn = functools.partial(_tc_loss_kernel, n_valid, br)
    return pl.pallas_call(
        kfn,
        grid=(nb,),
        in_specs=[
            pl.BlockSpec((br, H), lambda i: (i, 0)),
            pl.BlockSpec((br, 8), lambda i: (i, 0)),
            pl.BlockSpec(embed.shape, lambda i: (0, 0)),
            pl.BlockSpec(w_relation.shape, lambda i: (0, 0)),
        ],
        out_specs=pl.BlockSpec((1, 1), lambda i: (0, 0)),
        out_shape=jax.ShapeDtypeStruct((1, 1), jnp.float32),
    )(parts2, labels8, embed, w_relation)


def kernel(embed, src, rel, dst, labels, w_relation):
    e = src.shape[0]
    src = src.astype(jnp.int32)
    dst = dst.astype(jnp.int32)
    rel = rel.astype(jnp.int32)
    granule = 2 * 32 * CHUNK
    e_pad = ((e + granule - 1) // granule) * granule
    pad = e_pad - e
    src_p = jnp.pad(src, (0, pad))
    dst_p = jnp.pad(dst, (0, pad))
    rel_p = jnp.pad(rel, (0, pad))
    parts = _sc_parts(embed, src_p, dst_p, rel_p, w_relation)
    parts2 = parts.reshape(-1, H)
    labels8 = jnp.pad(labels, (0, pad)).reshape(-1, 8)
    loss = _tc_loss(e, parts2, labels8, embed, w_relation)
    return loss.reshape(())


# speedup vs baseline: 1.8740x; 1.0003x over previous
import functools

import jax
import jax.numpy as jnp
from jax import lax
from jax.experimental import pallas as pl
from jax.experimental.pallas import tpu as pltpu
from jax.experimental.pallas import tpu_sc as plsc

H = 128
LANES = 16
CHUNK = 128
NUM_R = 200
REG = 0.01


def _sc_parts_body(nc, ns, chunks_per_w,
                   embed_hbm, src_hbm, dst_hbm, rel_hbm, w_hbm,
                   out_hbm,
                   sidx0, didx0, ridx0, srows0, orows0, wrows0,
                   sidx1, didx1, ridx1, srows1, orows1, wrows1,
                   parts, sem0, sem1):
    wid = lax.axis_index("s") * nc + lax.axis_index("c")
    edges_per_w = chunks_per_w * CHUNK
    base_w = wid * edges_per_w
    bufs = ((sidx0, didx0, ridx0, srows0, orows0, wrows0, sem0),
            (sidx1, didx1, ridx1, srows1, orows1, wrows1, sem1))

    def fire(c, b):
        sidx, didx, ridx, srows, orows, wrows, sem = bufs[b]
        base = base_w + c * CHUNK
        pltpu.sync_copy(src_hbm.at[pl.ds(base, CHUNK)], sidx)
        pltpu.sync_copy(dst_hbm.at[pl.ds(base, CHUNK)], didx)
        pltpu.sync_copy(rel_hbm.at[pl.ds(base, CHUNK)], ridx)
        pltpu.async_copy(embed_hbm.at[sidx], srows, sem)
        pltpu.async_copy(embed_hbm.at[didx], orows, sem)
        pltpu.async_copy(w_hbm.at[ridx], wrows, sem)

    def drain(b):
        sidx, didx, ridx, srows, orows, wrows, sem = bufs[b]
        pltpu.make_async_copy(embed_hbm.at[sidx], srows, sem).wait()
        pltpu.make_async_copy(embed_hbm.at[didx], orows, sem).wait()
        pltpu.make_async_copy(w_hbm.at[ridx], wrows, sem).wait()

    def compute(c, b):
        sidx, didx, ridx, srows, orows, wrows, sem = bufs[b]
        base = base_w + c * CHUNK

        unroll = 4

        def edge_body(g, _):
            for u in range(unroll):
                e = g * unroll + u
                acc = (srows[e, pl.ds(0, LANES)] * orows[e, pl.ds(0, LANES)]
                       * wrows[e, pl.ds(0, LANES)])
                for j in range(1, H // LANES):
                    acc = acc + (srows[e, pl.ds(j * LANES, LANES)]
                                 * orows[e, pl.ds(j * LANES, LANES)]
                                 * wrows[e, pl.ds(j * LANES, LANES)])
                parts[e, pl.ds(0, LANES)] = acc
            return 0

        lax.fori_loop(0, CHUNK // unroll, edge_body, 0)
        pltpu.sync_copy(parts, out_hbm.at[pl.ds(base, CHUNK)])

    fire(0, 0)

    def pair_body(k, _):
        c = 2 * k
        fire(c + 1, 1)
        drain(0)
        compute(c, 0)

        @pl.when(c + 2 < chunks_per_w)
        def _():
            fire(c + 2, 0)

        drain(1)
        compute(c + 1, 1)
        return 0

    lax.fori_loop(0, chunks_per_w // 2, pair_body, 0)


def _sc_parts(embed, src, dst, rel, w_relation):
    e_pad = src.shape[0]
    info = plsc.get_sparse_core_info()
    nc, ns = info.num_cores, info.num_subcores
    nw = nc * ns
    chunks_per_w = e_pad // (nw * CHUNK)
    mesh = plsc.VectorSubcoreMesh(core_axis_name="c", subcore_axis_name="s")
    body = functools.partial(_sc_parts_body, nc, ns, chunks_per_w)
    buf_set = [
        pltpu.VMEM((CHUNK,), jnp.int32),
        pltpu.VMEM((CHUNK,), jnp.int32),
        pltpu.VMEM((CHUNK,), jnp.int32),
        pltpu.VMEM((CHUNK, H), jnp.float32),
        pltpu.VMEM((CHUNK, H), jnp.float32),
        pltpu.VMEM((CHUNK, H), jnp.float32),
    ]
    return pl.kernel(
        body,
        out_type=jax.ShapeDtypeStruct((e_pad, LANES), jnp.float32),
        mesh=mesh,
        scratch_types=buf_set + buf_set + [
            pltpu.VMEM((CHUNK, LANES), jnp.float32),
            pltpu.SemaphoreType.DMA,
            pltpu.SemaphoreType.DMA,
        ],
    )(embed, src, dst, rel, w_relation)


def _tc_loss_kernel(n_valid, br, parts_ref, labels_ref, embed_ref, w_ref,
                    out_ref):
    i = pl.program_id(0)

    @pl.when(i == 0)
    def _init():
        n_emb = embed_ref.shape[0] * embed_ref.shape[1]
        n_w = w_ref.shape[0] * w_ref.shape[1]
        reg = (jnp.sum(embed_ref[...] ** 2) / n_emb
               + jnp.sum(w_ref[...] ** 2) / n_w)
        out_ref[...] = jnp.reshape(REG * reg, (1, 1))

    p = parts_ref[...]
    d_iota = lax.broadcasted_iota(jnp.int32, (H, 8), 0)
    k_iota = lax.broadcasted_iota(jnp.int32, (H, 8), 1)
    fold = (d_iota // LANES == k_iota).astype(jnp.float32)
    s = jnp.dot(p, fold, preferred_element_type=jnp.float32)
    l = labels_ref[...]
    bce = jnp.maximum(s, 0.0) - s * l + jnp.log1p(jnp.exp(-jnp.abs(s)))
    row_i = lax.broadcasted_iota(jnp.int32, (br, 8), 0)
    k_i = lax.broadcasted_iota(jnp.int32, (br, 8), 1)
    edge_i = (i * br + row_i) * 8 + k_i
    bce = jnp.where(edge_i < n_valid, bce, 0.0)
    out_ref[...] += jnp.reshape(jnp.sum(bce) / n_valid, (1, 1))


def _tc_loss(n_valid, parts2, labels8, embed, w_relation):
    rows = parts2.shape[0]
    nb = 16
    while rows % nb:
        nb //= 2
    br = rows // nb
    kfn = functools.partial(_tc_loss_kernel, n_valid, br)
    return pl.pallas_call(
        kfn,
        grid=(nb,),
        in_specs=[
            pl.BlockSpec((br, H), lambda i: (i, 0)),
            pl.BlockSpec((br, 8), lambda i: (i, 0)),
            pl.BlockSpec(embed.shape, lambda i: (0, 0)),
            pl.BlockSpec(w_relation.shape, lambda i: (0, 0)),
        ],
        out_specs=pl.BlockSpec((1, 1), lambda i: (0, 0)),
        out_shape=jax.ShapeDtypeStruct((1, 1), jnp.float32),
    )(parts2, labels8, embed, w_relation)


def kernel(embed, src, rel, dst, labels, w_relation):
    e = src.shape[0]
    src = src.astype(jnp.int32)
    dst = dst.astype(jnp.int32)
    rel = rel.astype(jnp.int32)
    granule = 2 * 32 * CHUNK
    e_pad = ((e + granule - 1) // granule) * granule
    pad = e_pad - e
    src_p = jnp.pad(src, (0, pad))
    dst_p = jnp.pad(dst, (0, pad))
    rel_p = jnp.pad(rel, (0, pad))
    parts = _sc_parts(embed, src_p, dst_p, rel_p, w_relation)
    parts2 = parts.reshape(-1, H)
    labels8 = jnp.pad(labels, (0, pad)).reshape(-1, 8)
    loss = _tc_loss(e, parts2, labels8, embed, w_relation)
    return loss.reshape(())


# speedup vs baseline: 1.8899x; 1.0085x over previous
import functools

import jax
import jax.numpy as jnp
from jax import lax
from jax.experimental import pallas as pl
from jax.experimental.pallas import tpu as pltpu
from jax.experimental.pallas import tpu_sc as plsc

H = 128
LANES = 16
CHUNK = 128
NUM_R = 200
REG = 0.01


def _sc_parts_body(nc, ns, chunks_per_w,
                   embed_hbm, src_hbm, dst_hbm, rel_hbm, w_hbm,
                   out_hbm,
                   sidx0, didx0, ridx0, srows0, orows0, wrows0,
                   sidx1, didx1, ridx1, srows1, orows1, wrows1,
                   parts, sem0, sem1, semi0, semi1):
    wid = lax.axis_index("s") * nc + lax.axis_index("c")
    edges_per_w = chunks_per_w * CHUNK
    base_w = wid * edges_per_w
    bufs = ((sidx0, didx0, ridx0, srows0, orows0, wrows0, sem0, semi0),
            (sidx1, didx1, ridx1, srows1, orows1, wrows1, sem1, semi1))

    def fire_idx(c, b):
        sidx, didx, ridx, srows, orows, wrows, sem, semi = bufs[b]
        base = base_w + c * CHUNK
        pltpu.async_copy(src_hbm.at[pl.ds(base, CHUNK)], sidx, semi)
        pltpu.async_copy(dst_hbm.at[pl.ds(base, CHUNK)], didx, semi)
        pltpu.async_copy(rel_hbm.at[pl.ds(base, CHUNK)], ridx, semi)

    def fire_gather(c, b):
        sidx, didx, ridx, srows, orows, wrows, sem, semi = bufs[b]
        base = base_w + c * CHUNK
        pltpu.make_async_copy(src_hbm.at[pl.ds(base, CHUNK)], sidx, semi).wait()
        pltpu.make_async_copy(dst_hbm.at[pl.ds(base, CHUNK)], didx, semi).wait()
        pltpu.make_async_copy(rel_hbm.at[pl.ds(base, CHUNK)], ridx, semi).wait()
        pltpu.async_copy(embed_hbm.at[sidx], srows, sem)
        pltpu.async_copy(embed_hbm.at[didx], orows, sem)
        pltpu.async_copy(w_hbm.at[ridx], wrows, sem)

    def drain(b):
        sidx, didx, ridx, srows, orows, wrows, sem, semi = bufs[b]
        pltpu.make_async_copy(embed_hbm.at[sidx], srows, sem).wait()
        pltpu.make_async_copy(embed_hbm.at[didx], orows, sem).wait()
        pltpu.make_async_copy(w_hbm.at[ridx], wrows, sem).wait()

    def compute(c, b):
        sidx, didx, ridx, srows, orows, wrows, sem, semi = bufs[b]
        base = base_w + c * CHUNK

        unroll = 4

        def edge_body(g, _):
            for u in range(unroll):
                e = g * unroll + u
                acc = (srows[e, pl.ds(0, LANES)] * orows[e, pl.ds(0, LANES)]
                       * wrows[e, pl.ds(0, LANES)])
                for j in range(1, H // LANES):
                    acc = acc + (srows[e, pl.ds(j * LANES, LANES)]
                                 * orows[e, pl.ds(j * LANES, LANES)]
                                 * wrows[e, pl.ds(j * LANES, LANES)])
                parts[e, pl.ds(0, LANES)] = acc
            return 0

        lax.fori_loop(0, CHUNK // unroll, edge_body, 0)
        pltpu.sync_copy(parts, out_hbm.at[pl.ds(base, CHUNK)])

    fire_idx(0, 0)
    fire_idx(1, 1)
    fire_gather(0, 0)
    fire_gather(1, 1)

    def pair_body(k, _):
        c = 2 * k
        drain(0)

        @pl.when(c + 2 < chunks_per_w)
        def _():
            fire_idx(c + 2, 0)

        compute(c, 0)

        @pl.when(c + 2 < chunks_per_w)
        def _():
            fire_gather(c + 2, 0)

        drain(1)

        @pl.when(c + 3 < chunks_per_w)
        def _():
            fire_idx(c + 3, 1)

        compute(c + 1, 1)

        @pl.when(c + 3 < chunks_per_w)
        def _():
            fire_gather(c + 3, 1)

        return 0

    lax.fori_loop(0, chunks_per_w // 2, pair_body, 0)


def _sc_parts(embed, src, dst, rel, w_relation):
    e_pad = src.shape[0]
    info = plsc.get_sparse_core_info()
    nc, ns = info.num_cores, info.num_subcores
    nw = nc * ns
    chunks_per_w = e_pad // (nw * CHUNK)
    mesh = plsc.VectorSubcoreMesh(core_axis_name="c", subcore_axis_name="s")
    body = functools.partial(_sc_parts_body, nc, ns, chunks_per_w)
    buf_set = [
        pltpu.VMEM((CHUNK,), jnp.int32),
        pltpu.VMEM((CHUNK,), jnp.int32),
        pltpu.VMEM((CHUNK,), jnp.int32),
        pltpu.VMEM((CHUNK, H), jnp.float32),
        pltpu.VMEM((CHUNK, H), jnp.float32),
        pltpu.VMEM((CHUNK, H), jnp.float32),
    ]
    return pl.kernel(
        body,
        out_type=jax.ShapeDtypeStruct((e_pad, LANES), jnp.float32),
        mesh=mesh,
        scratch_types=buf_set + buf_set + [
            pltpu.VMEM((CHUNK, LANES), jnp.float32),
            pltpu.SemaphoreType.DMA,
            pltpu.SemaphoreType.DMA,
            pltpu.SemaphoreType.DMA,
            pltpu.SemaphoreType.DMA,
        ],
    )(embed, src, dst, rel, w_relation)


def _tc_loss_kernel(n_valid, br, parts_ref, labels_ref, embed_ref, w_ref,
                    out_ref):
    i = pl.program_id(0)

    @pl.when(i == 0)
    def _init():
        n_emb = embed_ref.shape[0] * embed_ref.shape[1]
        n_w = w_ref.shape[0] * w_ref.shape[1]
        reg = (jnp.sum(embed_ref[...] ** 2) / n_emb
               + jnp.sum(w_ref[...] ** 2) / n_w)
        out_ref[...] = jnp.reshape(REG * reg, (1, 1))

    p = parts_ref[...]
    d_iota = lax.broadcasted_iota(jnp.int32, (H, 8), 0)
    k_iota = lax.broadcasted_iota(jnp.int32, (H, 8), 1)
    fold = (d_iota // LANES == k_iota).astype(jnp.float32)
    s = jnp.dot(p, fold, preferred_element_type=jnp.float32)
    l = labels_ref[...]
    bce = jnp.maximum(s, 0.0) - s * l + jnp.log1p(jnp.exp(-jnp.abs(s)))
    row_i = lax.broadcasted_iota(jnp.int32, (br, 8), 0)
    k_i = lax.broadcasted_iota(jnp.int32, (br, 8), 1)
    edge_i = (i * br + row_i) * 8 + k_i
    bce = jnp.where(edge_i < n_valid, bce, 0.0)
    out_ref[...] += jnp.reshape(jnp.sum(bce) / n_valid, (1, 1))


def _tc_loss(n_valid, parts2, labels8, embed, w_relation):
    rows = parts2.shape[0]
    nb = 16
    while rows % nb:
        nb //= 2
    br = rows // nb
    kfn = functools.partial(_tc_loss_kernel, n_valid, br)
    return pl.pallas_call(
        kfn,
        grid=(nb,),
        in_specs=[
            pl.BlockSpec((br, H), lambda i: (i, 0)),
            pl.BlockSpec((br, 8), lambda i: (i, 0)),
            pl.BlockSpec(embed.shape, lambda i: (0, 0)),
            pl.BlockSpec(w_relation.shape, lambda i: (0, 0)),
        ],
        out_specs=pl.BlockSpec((1, 1), lambda i: (0, 0)),
        out_shape=jax.ShapeDtypeStruct((1, 1), jnp.float32),
    )(parts2, labels8, embed, w_relation)


def kernel(embed, src, rel, dst, labels, w_relation):
    e = src.shape[0]
    src = src.astype(jnp.int32)
    dst = dst.astype(jnp.int32)
    rel = rel.astype(jnp.int32)
    granule = 2 * 32 * CHUNK
    e_pad = ((e + granule - 1) // granule) * granule
    pad = e_pad - e
    src_p = jnp.pad(src, (0, pad))
    dst_p = jnp.pad(dst, (0, pad))
    rel_p = jnp.pad(rel, (0, pad))
    parts = _sc_parts(embed, src_p, dst_p, rel_p, w_relation)
    parts2 = parts.reshape(-1, H)
    labels8 = jnp.pad(labels, (0, pad)).reshape(-1, 8)
    loss = _tc_loss(e, parts2, labels8, embed, w_relation)
    return loss.reshape(())
